# Initial kernel scaffold; baseline (speedup 1.0000x reference)
import jax
import jax.numpy as jnp
from jax.experimental import pallas as pl


def kernel(x, edge_index, W_in, b_in, Wt1, bt1, Wt2, bt2, W_ih, W_hh, b_ih, b_hh, W_out, b_out):
    raise NotImplementedError("write your pallas kernel here")


# speedup vs baseline: 6.6095x; 6.6095x over previous
import functools

import jax
import jax.numpy as jnp
from jax import lax
from jax.experimental import pallas as pl
from jax.experimental.pallas import tpu as pltpu
from jax.experimental.pallas import tpu_sc as plsc

N = 10000
E = 320000
D = 128
MAX_REC = 5

NC = 2
NS = 16
NW = NC * NS
NPW = 320
NP = NW * NPW
CK = 128
EPAD = E + 2 * CK
OFFV = NPW + 16
OFFPAD = (NW - 1) * NPW + OFFV


def _sread(ref, i):
    return ref[pl.ds(i, 16)][0]


def _sc_agg_body(h_hbm, colp_hbm, rowp_hbm, offp_hbm, agg_hbm,
                 hl_v, agg_v, colbuf, cidx, ridx, off_v):
    c_id = lax.axis_index("c")
    s_id = lax.axis_index("s")
    wid = s_id * NC + c_id
    nbase = pl.multiple_of(wid * NPW, NPW)
    nrows = jnp.minimum(NPW, N - nbase)

    pltpu.sync_copy(offp_hbm.at[pl.ds(nbase, OFFV)], off_v)
    pltpu.sync_copy(h_hbm.at[pl.ds(nbase, NPW)], hl_v)

    zero16 = jnp.zeros((16,), jnp.float32)

    def zrow(r, carry):
        for k in range(8):
            agg_v[r, pl.ds(16 * k, 16)] = zero16
        return carry

    lax.fori_loop(0, NPW, zrow, 0)

    e0 = _sread(off_v, 0)
    e1 = _sread(off_v, nrows)
    abase = pl.multiple_of(e0 - lax.rem(e0, 8), 8)
    nch = lax.div(e1 - abase + CK - 1, CK)

    def chunk_body(c, carry):
        cbase = pl.multiple_of(abase + c * CK, 8)
        pltpu.sync_copy(colp_hbm.at[pl.ds(cbase, CK)], cidx)
        pltpu.sync_copy(rowp_hbm.at[pl.ds(cbase, CK)], ridx.at[pl.ds(0, CK)])
        pltpu.sync_copy(h_hbm.at[cidx], colbuf)
        cnt = jnp.minimum(CK, e1 - cbase)
        r_lo = jnp.maximum(_sread(ridx, 0), nbase)
        r_hi = jnp.minimum(_sread(ridx, cnt - 1), nbase + nrows - 1)

        def node_body(r, carry2):
            rl = r - nbase
            lo = jnp.maximum(_sread(off_v, rl), cbase)
            hi = jnp.minimum(_sread(off_v, rl + 1), cbase + cnt)
            hr = tuple(hl_v[rl, pl.ds(16 * k, 16)] for k in range(8))

            def edge_body(e, acc):
                j = e - cbase
                return tuple(
                    acc[k] + jnp.abs(hr[k] - colbuf[j, pl.ds(16 * k, 16)])
                    for k in range(8))

            acc0 = tuple(zero16 for _ in range(8))
            acc = lax.fori_loop(lo, hi, edge_body, acc0)
            for k in range(8):
                plsc.addupdate(agg_v.at[rl, pl.ds(16 * k, 16)], acc[k])
            return carry2

        lax.fori_loop(r_lo, r_hi + 1, node_body, 0)
        return carry

    lax.fori_loop(0, nch, chunk_body, 0)
    pltpu.sync_copy(agg_v, agg_hbm.at[pl.ds(nbase, NPW)])


def _make_sc_agg():
    mesh = plsc.VectorSubcoreMesh(core_axis_name="c", subcore_axis_name="s",
                                  num_cores=NC, num_subcores=NS)
    return pl.kernel(
        _sc_agg_body,
        out_type=jax.ShapeDtypeStruct((NP, D), jnp.float32),
        mesh=mesh,
        scratch_types=[
            pltpu.VMEM((NPW, D), jnp.float32),
            pltpu.VMEM((NPW, D), jnp.float32),
            pltpu.VMEM((CK, D), jnp.float32),
            pltpu.VMEM((CK,), jnp.int32),
            pltpu.VMEM((CK + 16,), jnp.int32),
            pltpu.VMEM((OFFV,), jnp.int32),
        ],
    )


BLK = 1280


def _proj_body(x_ref, w_ref, b_ref, o_ref, *, act):
    y = jnp.dot(x_ref[...], w_ref[...], preferred_element_type=jnp.float32)
    y = y + b_ref[...]
    if act == "relu":
        y = jnp.maximum(y, 0.0)
    o_ref[...] = y


def _proj(x, w_t, b, act):
    grid = (NP // BLK,)
    dout = w_t.shape[1]
    return pl.pallas_call(
        functools.partial(_proj_body, act=act),
        grid=grid,
        in_specs=[
            pl.BlockSpec((BLK, D), lambda i: (i, 0)),
            pl.BlockSpec((D, dout), lambda i: (0, 0)),
            pl.BlockSpec((1, dout), lambda i: (0, 0)),
        ],
        out_specs=pl.BlockSpec((BLK, dout), lambda i: (i, 0)),
        out_shape=jax.ShapeDtypeStruct((NP, dout), jnp.float32),
    )(x, w_t, b)


def _gru_body(agg_ref, h_ref, s_ref, wih_ref, whh_ref, bih_ref, bhh_ref,
              o_ref):
    agg = agg_ref[...]
    h = h_ref[...]
    gi = jnp.dot(agg, wih_ref[...], preferred_element_type=jnp.float32)
    gi = gi + bih_ref[...]
    gh = jnp.dot(h, whh_ref[...], preferred_element_type=jnp.float32)
    gh = gh + bhh_ref[...]
    r = jax.nn.sigmoid(gi[:, 0:D] + gh[:, 0:D])
    z = jax.nn.sigmoid(gi[:, D:2 * D] + gh[:, D:2 * D])
    n = jnp.tanh(gi[:, 2 * D:] + r * gh[:, 2 * D:])
    nh = (1.0 - z) * n + z * h
    mask = s_ref[...] > 0
    o_ref[...] = jnp.where(mask, nh, h)


def _gru(agg, h, steps, wih_t, whh_t, bih, bhh):
    grid = (NP // BLK,)
    return pl.pallas_call(
        _gru_body,
        grid=grid,
        in_specs=[
            pl.BlockSpec((BLK, D), lambda i: (i, 0)),
            pl.BlockSpec((BLK, D), lambda i: (i, 0)),
            pl.BlockSpec((BLK, 1), lambda i: (i, 0)),
            pl.BlockSpec((D, 3 * D), lambda i: (0, 0)),
            pl.BlockSpec((D, 3 * D), lambda i: (0, 0)),
            pl.BlockSpec((1, 3 * D), lambda i: (0, 0)),
            pl.BlockSpec((1, 3 * D), lambda i: (0, 0)),
        ],
        out_specs=pl.BlockSpec((BLK, D), lambda i: (i, 0)),
        out_shape=jax.ShapeDtypeStruct((NP, D), jnp.float32),
    )(agg, h, steps, wih_t, whh_t, bih, bhh)


def kernel(x, edge_index, W_in, b_in, Wt1, bt1, Wt2, bt2,
           W_ih, W_hh, b_ih, b_hh, W_out, b_out):
    x = x.astype(jnp.float32)
    row = edge_index[0].astype(jnp.int32)
    col = edge_index[1].astype(jnp.int32)

    row_s, col_s = lax.sort_key_val(row, col)
    offsets = jnp.searchsorted(
        row_s, jnp.arange(N + 1, dtype=jnp.int32), side="left"
    ).astype(jnp.int32)
    colp = jnp.concatenate(
        [col_s, jnp.zeros((EPAD - E,), jnp.int32)])
    rowp = jnp.concatenate(
        [row_s, jnp.full((EPAD - E,), N - 1, jnp.int32)])
    offp = jnp.concatenate(
        [offsets, jnp.full((OFFPAD - (N + 1),), E, jnp.int32)])

    t = jax.nn.relu(x @ Wt1.T + bt1)
    tau_val = jax.nn.softplus(t @ Wt2.T + bt2).squeeze()
    steps0 = jnp.minimum((1.0 / tau_val).astype(jnp.int32), MAX_REC)
    steps_p = jnp.concatenate(
        [steps0, jnp.zeros((NP - N,), jnp.int32)]).reshape(NP, 1)

    x_p = jnp.concatenate([x, jnp.zeros((NP - N, D), jnp.float32)])

    w_in_t = W_in.T
    wih_t = W_ih.T
    whh_t = W_hh.T
    w_out_t = W_out.T
    b_in2 = b_in.reshape(1, D)
    bih2 = b_ih.reshape(1, 3 * D)
    bhh2 = b_hh.reshape(1, 3 * D)
    b_out2 = b_out.reshape(1, D)

    h0 = _proj(x_p, w_in_t, b_in2, "relu")

    sc_agg = _make_sc_agg()

    def cond(carry):
        _, steps, i = carry
        return jnp.logical_and(i < MAX_REC, jnp.max(steps) > 0)

    def body(carry):
        h, steps, i = carry
        agg = sc_agg(h, colp, rowp, offp)
        h2 = _gru(agg, h, steps, wih_t, whh_t, bih2, bhh2)
        return h2, steps - 1, i + 1

    h, _, _ = lax.while_loop(cond, body, (h0, steps_p, jnp.int32(0)))

    out = _proj(h, w_out_t, b_out2, "none")
    return out[:N]


# speedup vs baseline: 10.0536x; 1.5211x over previous
import functools

import jax
import jax.numpy as jnp
from jax import lax
from jax.experimental import pallas as pl
from jax.experimental.pallas import tpu as pltpu
from jax.experimental.pallas import tpu_sc as plsc

N = 10000
E = 320000
D = 128
MAX_REC = 5

NC = 2
NS = 16
NW = NC * NS
NPW = 320
NP = NW * NPW
CK = 128
EPAD = E + 2 * CK
OFFV = NPW + 16
OFFPAD = (NW - 1) * NPW + OFFV


def _sread(ref, i):
    return ref[pl.ds(i, 16)][0]


EPW = 10240
HCK = 1280
ETOT = NW * EPW
NH = 10032
NHC = 10008


def _sc_hist_body(row_hbm, counts_hbm, ridx_v, cnt_v):
    c_id = lax.axis_index("c")
    s_id = lax.axis_index("s")
    wid = s_id * NC + c_id
    ebase = pl.multiple_of(wid * EPW, EPW)
    zero16 = jnp.zeros((16,), jnp.int32)
    ones16 = jnp.ones((16,), jnp.int32)

    for g in range(NH // 16):
        cnt_v[pl.ds(16 * g, 16)] = zero16

    def chunk(c, carry):
        pltpu.sync_copy(row_hbm.at[pl.ds(ebase + c * HCK, HCK)], ridx_v)

        def gbody(g, carry2):
            rows16 = ridx_v[pl.ds(16 * g, 16)]
            plsc.addupdate_scatter(cnt_v, [rows16], ones16,
                                   mask=jnp.ones((16,), jnp.bool_))
            return carry2

        lax.fori_loop(0, HCK // 16, gbody, 0)
        return carry

    lax.fori_loop(0, EPW // HCK, chunk, 0)
    pltpu.sync_copy(cnt_v.at[pl.ds(0, NHC)],
                    counts_hbm.at[pl.ds(pl.multiple_of(wid * NHC, 8), NHC)])


def _sc_bin_body(row_hbm, col_hbm, bases_hbm, rows_out, cols_out,
                 ridx_v, cidx_v, cur_v, pos_v, sem):
    c_id = lax.axis_index("c")
    s_id = lax.axis_index("s")
    wid = s_id * NC + c_id
    ebase = pl.multiple_of(wid * EPW, EPW)
    iota16 = lax.iota(jnp.int32, 16)
    ohs = [(iota16 == j).astype(jnp.int32) for j in range(16)]

    pltpu.sync_copy(
        bases_hbm.at[pl.ds(pl.multiple_of(wid * NHC, 8), NHC)],
        cur_v.at[pl.ds(0, NHC)])
    lane0 = lax.iota(jnp.int32, 16) == 0
    ones16 = jnp.ones((16,), jnp.int32)

    def chunk(c, carry):
        pltpu.sync_copy(row_hbm.at[pl.ds(ebase + c * HCK, HCK)], ridx_v)
        pltpu.sync_copy(col_hbm.at[pl.ds(ebase + c * HCK, HCK)], cidx_v)

        def gbody(g, carry2):
            kk = lax.div(g, 8)
            m = lax.rem(g, 8)
            posacc = jnp.zeros((16,), jnp.int32)
            for j in range(16):
                r = _sread(ridx_v, g * 16 + j)
                rs = jnp.full((16,), r, jnp.int32)
                v = plsc.load_gather(cur_v, [rs])
                plsc.addupdate_scatter(cur_v, [rs], ones16, mask=lane0)
                posacc = posacc + ohs[j] * v[0]
            pos_v[kk, pl.ds(16 * m, 16)] = posacc
            return carry2

        lax.fori_loop(0, HCK // 16, gbody, 0)
        for k in range(HCK // 128):
            pltpu.async_copy(cidx_v.at[pl.ds(128 * k, 128)],
                             cols_out.at[pos_v.at[k]], sem).wait()
            pltpu.async_copy(ridx_v.at[pl.ds(128 * k, 128)],
                             rows_out.at[pos_v.at[k]], sem).wait()
        return carry

    lax.fori_loop(0, EPW // HCK, chunk, 0)


def _make_sc_hist():
    mesh = plsc.VectorSubcoreMesh(core_axis_name="c", subcore_axis_name="s",
                                  num_cores=NC, num_subcores=NS)
    return pl.kernel(
        _sc_hist_body,
        out_type=jax.ShapeDtypeStruct((NW * NHC,), jnp.int32),
        mesh=mesh,
        scratch_types=[
            pltpu.VMEM((HCK,), jnp.int32),
            pltpu.VMEM((NH,), jnp.int32),
        ],
        compiler_params=pltpu.CompilerParams(needs_layout_passes=False),
    )


def _make_sc_bin():
    mesh = plsc.VectorSubcoreMesh(core_axis_name="c", subcore_axis_name="s",
                                  num_cores=NC, num_subcores=NS)
    return pl.kernel(
        _sc_bin_body,
        out_type=(jax.ShapeDtypeStruct((ETOT,), jnp.int32),
                  jax.ShapeDtypeStruct((ETOT,), jnp.int32)),
        mesh=mesh,
        scratch_types=[
            pltpu.VMEM((HCK,), jnp.int32),
            pltpu.VMEM((HCK,), jnp.int32),
            pltpu.VMEM((NH,), jnp.int32),
            pltpu.VMEM((HCK // 128, 128), jnp.int32),
            pltpu.SemaphoreType.DMA,
        ],
        compiler_params=pltpu.CompilerParams(needs_layout_passes=False),
    )


def _sc_agg_body(h_hbm, colp_hbm, rowp_hbm, offp_hbm, agg_hbm,
                 hl_v, agg_v, colbuf, cidx, ridx, off_v):
    c_id = lax.axis_index("c")
    s_id = lax.axis_index("s")
    wid = s_id * NC + c_id
    nbase = pl.multiple_of(wid * NPW, NPW)
    nrows = jnp.minimum(NPW, N - nbase)

    pltpu.sync_copy(offp_hbm.at[pl.ds(nbase, OFFV)], off_v)
    pltpu.sync_copy(h_hbm.at[pl.ds(nbase, NPW)], hl_v)

    zero16 = jnp.zeros((16,), jnp.float32)

    def zrow(r, carry):
        for k in range(8):
            agg_v[r, pl.ds(16 * k, 16)] = zero16
        return carry

    lax.fori_loop(0, NPW, zrow, 0)

    e0 = _sread(off_v, 0)
    e1 = _sread(off_v, nrows)
    abase = pl.multiple_of(e0 - lax.rem(e0, 8), 8)
    nch = lax.div(e1 - abase + CK - 1, CK)

    def chunk_body(c, carry):
        cbase = pl.multiple_of(abase + c * CK, 8)
        pltpu.sync_copy(colp_hbm.at[pl.ds(cbase, CK)], cidx)
        pltpu.sync_copy(rowp_hbm.at[pl.ds(cbase, CK)], ridx.at[pl.ds(0, CK)])
        pltpu.sync_copy(h_hbm.at[cidx], colbuf)
        cnt = jnp.minimum(CK, e1 - cbase)
        r_lo = jnp.maximum(_sread(ridx, 0), nbase)
        r_hi = jnp.minimum(_sread(ridx, cnt - 1), nbase + nrows - 1)

        def node_body(r, carry2):
            rl = r - nbase
            lo = jnp.maximum(_sread(off_v, rl), cbase)
            hi = jnp.minimum(_sread(off_v, rl + 1), cbase + cnt)
            hr = tuple(hl_v[rl, pl.ds(16 * k, 16)] for k in range(8))

            def edge_body(e, acc):
                j = e - cbase
                return tuple(
                    acc[k] + jnp.abs(hr[k] - colbuf[j, pl.ds(16 * k, 16)])
                    for k in range(8))

            acc0 = tuple(zero16 for _ in range(8))
            acc = lax.fori_loop(lo, hi, edge_body, acc0)
            for k in range(8):
                plsc.addupdate(agg_v.at[rl, pl.ds(16 * k, 16)], acc[k])
            return carry2

        lax.fori_loop(r_lo, r_hi + 1, node_body, 0)
        return carry

    lax.fori_loop(0, nch, chunk_body, 0)
    pltpu.sync_copy(agg_v, agg_hbm.at[pl.ds(nbase, NPW)])


def _make_sc_agg():
    mesh = plsc.VectorSubcoreMesh(core_axis_name="c", subcore_axis_name="s",
                                  num_cores=NC, num_subcores=NS)
    return pl.kernel(
        _sc_agg_body,
        out_type=jax.ShapeDtypeStruct((NP, D), jnp.float32),
        mesh=mesh,
        scratch_types=[
            pltpu.VMEM((NPW, D), jnp.float32),
            pltpu.VMEM((NPW, D), jnp.float32),
            pltpu.VMEM((CK, D), jnp.float32),
            pltpu.VMEM((CK,), jnp.int32),
            pltpu.VMEM((CK + 16,), jnp.int32),
            pltpu.VMEM((OFFV,), jnp.int32),
        ],
    )


BLK = 1280


def _proj_body(x_ref, w_ref, b_ref, o_ref, *, act):
    y = jnp.dot(x_ref[...], w_ref[...], preferred_element_type=jnp.float32)
    y = y + b_ref[...]
    if act == "relu":
        y = jnp.maximum(y, 0.0)
    o_ref[...] = y


def _proj(x, w_t, b, act):
    grid = (NP // BLK,)
    dout = w_t.shape[1]
    return pl.pallas_call(
        functools.partial(_proj_body, act=act),
        grid=grid,
        in_specs=[
            pl.BlockSpec((BLK, D), lambda i: (i, 0)),
            pl.BlockSpec((D, dout), lambda i: (0, 0)),
            pl.BlockSpec((1, dout), lambda i: (0, 0)),
        ],
        out_specs=pl.BlockSpec((BLK, dout), lambda i: (i, 0)),
        out_shape=jax.ShapeDtypeStruct((NP, dout), jnp.float32),
    )(x, w_t, b)


def _gru_body(agg_ref, h_ref, s_ref, wih_ref, whh_ref, bih_ref, bhh_ref,
              o_ref):
    agg = agg_ref[...]
    h = h_ref[...]
    gi = jnp.dot(agg, wih_ref[...], preferred_element_type=jnp.float32)
    gi = gi + bih_ref[...]
    gh = jnp.dot(h, whh_ref[...], preferred_element_type=jnp.float32)
    gh = gh + bhh_ref[...]
    r = jax.nn.sigmoid(gi[:, 0:D] + gh[:, 0:D])
    z = jax.nn.sigmoid(gi[:, D:2 * D] + gh[:, D:2 * D])
    n = jnp.tanh(gi[:, 2 * D:] + r * gh[:, 2 * D:])
    nh = (1.0 - z) * n + z * h
    mask = s_ref[...] > 0
    o_ref[...] = jnp.where(mask, nh, h)


def _gru(agg, h, steps, wih_t, whh_t, bih, bhh):
    grid = (NP // BLK,)
    return pl.pallas_call(
        _gru_body,
        grid=grid,
        in_specs=[
            pl.BlockSpec((BLK, D), lambda i: (i, 0)),
            pl.BlockSpec((BLK, D), lambda i: (i, 0)),
            pl.BlockSpec((BLK, 1), lambda i: (i, 0)),
            pl.BlockSpec((D, 3 * D), lambda i: (0, 0)),
            pl.BlockSpec((D, 3 * D), lambda i: (0, 0)),
            pl.BlockSpec((1, 3 * D), lambda i: (0, 0)),
            pl.BlockSpec((1, 3 * D), lambda i: (0, 0)),
        ],
        out_specs=pl.BlockSpec((BLK, D), lambda i: (i, 0)),
        out_shape=jax.ShapeDtypeStruct((NP, D), jnp.float32),
    )(agg, h, steps, wih_t, whh_t, bih, bhh)


def kernel(x, edge_index, W_in, b_in, Wt1, bt1, Wt2, bt2,
           W_ih, W_hh, b_ih, b_hh, W_out, b_out):
    x = x.astype(jnp.float32)
    row = edge_index[0].astype(jnp.int32)
    col = edge_index[1].astype(jnp.int32)

    rowp32 = jnp.concatenate([row, jnp.full((ETOT - E,), N, jnp.int32)])
    colp32 = jnp.concatenate([col, jnp.zeros((ETOT - E,), jnp.int32)])
    counts = _make_sc_hist()(rowp32).reshape(NW, NHC)
    cnts = counts[:, :N + 1]
    flat = cnts.T.reshape(-1)
    basesF = jnp.cumsum(flat) - flat
    bases2 = basesF.reshape(N + 1, NW).astype(jnp.int32)
    offsets = bases2[:, 0]
    basesT = jnp.concatenate(
        [bases2.T, jnp.zeros((NW, NHC - (N + 1)), jnp.int32)],
        axis=1).reshape(-1)
    rowp, colp = _make_sc_bin()(rowp32, colp32, basesT)
    offp = jnp.concatenate(
        [offsets, jnp.full((OFFPAD - (N + 1),), E, jnp.int32)])

    t = jax.nn.relu(x @ Wt1.T + bt1)
    tau_val = jax.nn.softplus(t @ Wt2.T + bt2).squeeze()
    steps0 = jnp.minimum((1.0 / tau_val).astype(jnp.int32), MAX_REC)
    steps_p = jnp.concatenate(
        [steps0, jnp.zeros((NP - N,), jnp.int32)]).reshape(NP, 1)

    x_p = jnp.concatenate([x, jnp.zeros((NP - N, D), jnp.float32)])

    w_in_t = W_in.T
    wih_t = W_ih.T
    whh_t = W_hh.T
    w_out_t = W_out.T
    b_in2 = b_in.reshape(1, D)
    bih2 = b_ih.reshape(1, 3 * D)
    bhh2 = b_hh.reshape(1, 3 * D)
    b_out2 = b_out.reshape(1, D)

    h0 = _proj(x_p, w_in_t, b_in2, "relu")

    sc_agg = _make_sc_agg()

    def cond(carry):
        _, steps, i = carry
        return jnp.logical_and(i < MAX_REC, jnp.max(steps) > 0)

    def body(carry):
        h, steps, i = carry
        agg = sc_agg(h, colp, rowp, offp)
        h2 = _gru(agg, h, steps, wih_t, whh_t, bih2, bhh2)
        return h2, steps - 1, i + 1

    h, _, _ = lax.while_loop(cond, body, (h0, steps_p, jnp.int32(0)))

    out = _proj(h, w_out_t, b_out2, "none")
    return out[:N]


# speedup vs baseline: 11.1444x; 1.1085x over previous
import functools

import jax
import jax.numpy as jnp
from jax import lax
from jax.experimental import pallas as pl
from jax.experimental.pallas import tpu as pltpu
from jax.experimental.pallas import tpu_sc as plsc

N = 10000
E = 320000
D = 128
MAX_REC = 5

NC = 2
NS = 16
NW = NC * NS
NPW = 320
NP = NW * NPW
CK = 128
EPAD = E + 2 * CK
OFFV = NPW + 16
OFFPAD = (NW - 1) * NPW + OFFV


def _sread(ref, i):
    return ref[pl.ds(i, 16)][0]


EPW = 10240
HCK = 1280
ETOT = NW * EPW
NH = 10032
NHC = 10008


def _sc_hist_body(row_hbm, counts_hbm, ridx_v, cnt_v):
    c_id = lax.axis_index("c")
    s_id = lax.axis_index("s")
    wid = s_id * NC + c_id
    ebase = pl.multiple_of(wid * EPW, EPW)
    zero16 = jnp.zeros((16,), jnp.int32)
    ones16 = jnp.ones((16,), jnp.int32)

    for g in range(NH // 16):
        cnt_v[pl.ds(16 * g, 16)] = zero16

    def chunk(c, carry):
        pltpu.sync_copy(row_hbm.at[pl.ds(ebase + c * HCK, HCK)], ridx_v)

        def gbody(g, carry2):
            rows16 = ridx_v[pl.ds(16 * g, 16)]
            plsc.addupdate_scatter(cnt_v, [rows16], ones16,
                                   mask=jnp.ones((16,), jnp.bool_))
            return carry2

        lax.fori_loop(0, HCK // 16, gbody, 0)
        return carry

    lax.fori_loop(0, EPW // HCK, chunk, 0)
    pltpu.sync_copy(cnt_v.at[pl.ds(0, NHC)],
                    counts_hbm.at[pl.ds(pl.multiple_of(wid * NHC, 8), NHC)])


def _sc_bin_body(row_hbm, col_hbm, bases_hbm, rows_out, cols_out,
                 ridx_v, cidx_v, cur_v, pos_v, sem):
    c_id = lax.axis_index("c")
    s_id = lax.axis_index("s")
    wid = s_id * NC + c_id
    ebase = pl.multiple_of(wid * EPW, EPW)
    iota16 = lax.iota(jnp.int32, 16)
    ohs = [(iota16 == j).astype(jnp.int32) for j in range(16)]

    pltpu.sync_copy(
        bases_hbm.at[pl.ds(pl.multiple_of(wid * NHC, 8), NHC)],
        cur_v.at[pl.ds(0, NHC)])
    lane0 = lax.iota(jnp.int32, 16) == 0
    ones16 = jnp.ones((16,), jnp.int32)

    def chunk(c, carry):
        pltpu.sync_copy(row_hbm.at[pl.ds(ebase + c * HCK, HCK)], ridx_v)
        pltpu.sync_copy(col_hbm.at[pl.ds(ebase + c * HCK, HCK)], cidx_v)

        def gbody(g, carry2):
            kk = lax.div(g, 8)
            m = lax.rem(g, 8)
            rows16 = ridx_v[pl.ds(16 * g, 16)]
            posacc = jnp.zeros((16,), jnp.int32)
            for j in range(16):
                rs = jnp.full((16,), 0, jnp.int32) + rows16[j]
                v = plsc.load_gather(cur_v, [rs])
                plsc.addupdate_scatter(cur_v, [rs], ones16, mask=lane0)
                posacc = posacc + ohs[j] * v
            pos_v[kk, pl.ds(16 * m, 16)] = posacc
            return carry2

        lax.fori_loop(0, HCK // 16, gbody, 0)
        cps = []
        for k in range(HCK // 128):
            cps.append(pltpu.async_copy(cidx_v.at[pl.ds(128 * k, 128)],
                                        cols_out.at[pos_v.at[k]], sem))
            cps.append(pltpu.async_copy(ridx_v.at[pl.ds(128 * k, 128)],
                                        rows_out.at[pos_v.at[k]], sem))
        for cp in cps:
            cp.wait()
        return carry

    lax.fori_loop(0, EPW // HCK, chunk, 0)


def _make_sc_hist():
    mesh = plsc.VectorSubcoreMesh(core_axis_name="c", subcore_axis_name="s",
                                  num_cores=NC, num_subcores=NS)
    return pl.kernel(
        _sc_hist_body,
        out_type=jax.ShapeDtypeStruct((NW * NHC,), jnp.int32),
        mesh=mesh,
        scratch_types=[
            pltpu.VMEM((HCK,), jnp.int32),
            pltpu.VMEM((NH,), jnp.int32),
        ],
        compiler_params=pltpu.CompilerParams(needs_layout_passes=False),
    )


def _make_sc_bin():
    mesh = plsc.VectorSubcoreMesh(core_axis_name="c", subcore_axis_name="s",
                                  num_cores=NC, num_subcores=NS)
    return pl.kernel(
        _sc_bin_body,
        out_type=(jax.ShapeDtypeStruct((ETOT,), jnp.int32),
                  jax.ShapeDtypeStruct((ETOT,), jnp.int32)),
        mesh=mesh,
        scratch_types=[
            pltpu.VMEM((HCK,), jnp.int32),
            pltpu.VMEM((HCK,), jnp.int32),
            pltpu.VMEM((NH,), jnp.int32),
            pltpu.VMEM((HCK // 128, 128), jnp.int32),
            pltpu.SemaphoreType.DMA,
        ],
        compiler_params=pltpu.CompilerParams(needs_layout_passes=False),
    )


def _sc_agg_body(h_hbm, colp_hbm, rowp_hbm, offp_hbm, agg_hbm,
                 hl_v, agg_v, colbuf, cidx, ridx0, ridx1, off_v, sem0, sem1):
    c_id = lax.axis_index("c")
    s_id = lax.axis_index("s")
    wid = s_id * NC + c_id
    nbase = pl.multiple_of(wid * NPW, NPW)
    nrows = jnp.minimum(NPW, N - nbase)

    pltpu.sync_copy(offp_hbm.at[pl.ds(nbase, OFFV)], off_v)
    pltpu.sync_copy(h_hbm.at[pl.ds(nbase, NPW)], hl_v)

    zero16 = jnp.zeros((16,), jnp.float32)

    def zrow(r, carry):
        for k in range(8):
            agg_v[r, pl.ds(16 * k, 16)] = zero16
        return carry

    lax.fori_loop(0, NPW, zrow, 0)

    e0 = _sread(off_v, 0)
    e1 = _sread(off_v, nrows)
    abase = pl.multiple_of(e0 - lax.rem(e0, 8), 8)
    nch = lax.div(e1 - abase + CK - 1, CK)
    sems = (sem0, sem1)
    ridxs = (ridx0, ridx1)

    def issue(c, b):
        cbase = pl.multiple_of(abase + c * CK, 8)
        pltpu.sync_copy(colp_hbm.at[pl.ds(cbase, CK)], cidx.at[b])
        pltpu.sync_copy(rowp_hbm.at[pl.ds(cbase, CK)], ridxs[b].at[pl.ds(0, CK)])
        pltpu.async_copy(h_hbm.at[cidx.at[b]], colbuf.at[b], sems[b])

    def compute(c, b):
        cbase = pl.multiple_of(abase + c * CK, 8)
        cnt = jnp.minimum(CK, e1 - cbase)
        rb = ridxs[b]
        r_lo = jnp.maximum(_sread(rb, 0), nbase)
        r_hi = jnp.minimum(_sread(rb, cnt - 1), nbase + nrows - 1)

        def node_body(r, carry2):
            rl = r - nbase
            lo = jnp.maximum(_sread(off_v, rl), cbase)
            hi = jnp.minimum(_sread(off_v, rl + 1), cbase + cnt)
            hr = tuple(hl_v[rl, pl.ds(16 * k, 16)] for k in range(8))

            def edge_body(e, acc):
                j = e - cbase
                return tuple(
                    acc[k] + jnp.abs(hr[k] - colbuf[b, j, pl.ds(16 * k, 16)])
                    for k in range(8))

            acc0 = tuple(zero16 for _ in range(8))
            acc = lax.fori_loop(lo, hi, edge_body, acc0)
            for k in range(8):
                plsc.addupdate(agg_v.at[rl, pl.ds(16 * k, 16)], acc[k])
            return carry2

        lax.fori_loop(r_lo, r_hi + 1, node_body, 0)

    @pl.when(nch > 0)
    def _():
        issue(0, 0)

    def pair_body(p, carry):
        for b in range(2):
            c = 2 * p + b

            @pl.when(c < nch)
            def _():
                @pl.when(c + 1 < nch)
                def _():
                    issue(c + 1, 1 - b)

                pltpu.make_async_copy(h_hbm.at[cidx.at[b]],
                                      colbuf.at[b], sems[b]).wait()
                compute(c, b)
        return carry

    lax.fori_loop(0, lax.div(nch + 1, 2), pair_body, 0)
    pltpu.sync_copy(agg_v, agg_hbm.at[pl.ds(nbase, NPW)])


def _make_sc_agg():
    mesh = plsc.VectorSubcoreMesh(core_axis_name="c", subcore_axis_name="s",
                                  num_cores=NC, num_subcores=NS)
    return pl.kernel(
        _sc_agg_body,
        out_type=jax.ShapeDtypeStruct((NP, D), jnp.float32),
        mesh=mesh,
        scratch_types=[
            pltpu.VMEM((NPW, D), jnp.float32),
            pltpu.VMEM((NPW, D), jnp.float32),
            pltpu.VMEM((2, CK, D), jnp.float32),
            pltpu.VMEM((2, CK), jnp.int32),
            pltpu.VMEM((CK + 16,), jnp.int32),
            pltpu.VMEM((CK + 16,), jnp.int32),
            pltpu.VMEM((OFFV,), jnp.int32),
            pltpu.SemaphoreType.DMA,
            pltpu.SemaphoreType.DMA,
        ],
    )


BLK = 1280


def _proj_body(x_ref, w_ref, b_ref, o_ref, *, act):
    y = jnp.dot(x_ref[...], w_ref[...], preferred_element_type=jnp.float32)
    y = y + b_ref[...]
    if act == "relu":
        y = jnp.maximum(y, 0.0)
    o_ref[...] = y


def _proj(x, w_t, b, act):
    grid = (NP // BLK,)
    dout = w_t.shape[1]
    return pl.pallas_call(
        functools.partial(_proj_body, act=act),
        grid=grid,
        in_specs=[
            pl.BlockSpec((BLK, D), lambda i: (i, 0)),
            pl.BlockSpec((D, dout), lambda i: (0, 0)),
            pl.BlockSpec((1, dout), lambda i: (0, 0)),
        ],
        out_specs=pl.BlockSpec((BLK, dout), lambda i: (i, 0)),
        out_shape=jax.ShapeDtypeStruct((NP, dout), jnp.float32),
    )(x, w_t, b)


def _gru_body(agg_ref, h_ref, s_ref, wih_ref, whh_ref, bih_ref, bhh_ref,
              o_ref):
    agg = agg_ref[...]
    h = h_ref[...]
    gi = jnp.dot(agg, wih_ref[...], preferred_element_type=jnp.float32)
    gi = gi + bih_ref[...]
    gh = jnp.dot(h, whh_ref[...], preferred_element_type=jnp.float32)
    gh = gh + bhh_ref[...]
    r = jax.nn.sigmoid(gi[:, 0:D] + gh[:, 0:D])
    z = jax.nn.sigmoid(gi[:, D:2 * D] + gh[:, D:2 * D])
    n = jnp.tanh(gi[:, 2 * D:] + r * gh[:, 2 * D:])
    nh = (1.0 - z) * n + z * h
    mask = s_ref[...] > 0
    o_ref[...] = jnp.where(mask, nh, h)


def _gru(agg, h, steps, wih_t, whh_t, bih, bhh):
    grid = (NP // BLK,)
    return pl.pallas_call(
        _gru_body,
        grid=grid,
        in_specs=[
            pl.BlockSpec((BLK, D), lambda i: (i, 0)),
            pl.BlockSpec((BLK, D), lambda i: (i, 0)),
            pl.BlockSpec((BLK, 1), lambda i: (i, 0)),
            pl.BlockSpec((D, 3 * D), lambda i: (0, 0)),
            pl.BlockSpec((D, 3 * D), lambda i: (0, 0)),
            pl.BlockSpec((1, 3 * D), lambda i: (0, 0)),
            pl.BlockSpec((1, 3 * D), lambda i: (0, 0)),
        ],
        out_specs=pl.BlockSpec((BLK, D), lambda i: (i, 0)),
        out_shape=jax.ShapeDtypeStruct((NP, D), jnp.float32),
    )(agg, h, steps, wih_t, whh_t, bih, bhh)


def kernel(x, edge_index, W_in, b_in, Wt1, bt1, Wt2, bt2,
           W_ih, W_hh, b_ih, b_hh, W_out, b_out):
    x = x.astype(jnp.float32)
    row = edge_index[0].astype(jnp.int32)
    col = edge_index[1].astype(jnp.int32)

    rowp32 = jnp.concatenate([row, jnp.full((ETOT - E,), N, jnp.int32)])
    colp32 = jnp.concatenate([col, jnp.zeros((ETOT - E,), jnp.int32)])
    counts = _make_sc_hist()(rowp32).reshape(NW, NHC)
    cnts = counts[:, :N + 1]
    flat = cnts.T.reshape(-1)
    basesF = jnp.cumsum(flat) - flat
    bases2 = basesF.reshape(N + 1, NW).astype(jnp.int32)
    offsets = bases2[:, 0]
    basesT = jnp.concatenate(
        [bases2.T, jnp.zeros((NW, NHC - (N + 1)), jnp.int32)],
        axis=1).reshape(-1)
    rowp, colp = _make_sc_bin()(rowp32, colp32, basesT)
    offp = jnp.concatenate(
        [offsets, jnp.full((OFFPAD - (N + 1),), E, jnp.int32)])

    t = jax.nn.relu(x @ Wt1.T + bt1)
    tau_val = jax.nn.softplus(t @ Wt2.T + bt2).squeeze()
    steps0 = jnp.minimum((1.0 / tau_val).astype(jnp.int32), MAX_REC)
    steps_p = jnp.concatenate(
        [steps0, jnp.zeros((NP - N,), jnp.int32)]).reshape(NP, 1)

    x_p = jnp.concatenate([x, jnp.zeros((NP - N, D), jnp.float32)])

    w_in_t = W_in.T
    wih_t = W_ih.T
    whh_t = W_hh.T
    w_out_t = W_out.T
    b_in2 = b_in.reshape(1, D)
    bih2 = b_ih.reshape(1, 3 * D)
    bhh2 = b_hh.reshape(1, 3 * D)
    b_out2 = b_out.reshape(1, D)

    h0 = _proj(x_p, w_in_t, b_in2, "relu")

    sc_agg = _make_sc_agg()

    def cond(carry):
        _, steps, i = carry
        return jnp.logical_and(i < MAX_REC, jnp.max(steps) > 0)

    def body(carry):
        h, steps, i = carry
        agg = sc_agg(h, colp, rowp, offp)
        h2 = _gru(agg, h, steps, wih_t, whh_t, bih2, bhh2)
        return h2, steps - 1, i + 1

    h, _, _ = lax.while_loop(cond, body, (h0, steps_p, jnp.int32(0)))

    out = _proj(h, w_out_t, b_out2, "none")
    return out[:N]


# speedup vs baseline: 11.1544x; 1.0009x over previous
import functools

import jax
import jax.numpy as jnp
from jax import lax
from jax.experimental import pallas as pl
from jax.experimental.pallas import tpu as pltpu
from jax.experimental.pallas import tpu_sc as plsc

N = 10000
E = 320000
D = 128
MAX_REC = 5

NC = 2
NS = 16
NW = NC * NS
NPW = 320
NP = NW * NPW
CK = 128
EPAD = E + 2 * CK
OFFV = NPW + 16
OFFPAD = (NW - 1) * NPW + OFFV


def _sread(ref, i):
    return ref[pl.ds(i, 16)][0]


EPW = 10240
HCK = 1280
ETOT = NW * EPW
NH = 10032
NHC = 10008


def _sc_hist_body(row_hbm, counts_hbm, ridx_v, cnt_v):
    c_id = lax.axis_index("c")
    s_id = lax.axis_index("s")
    wid = s_id * NC + c_id
    ebase = pl.multiple_of(wid * EPW, EPW)
    zero16 = jnp.zeros((16,), jnp.int32)
    ones16 = jnp.ones((16,), jnp.int32)

    for g in range(NH // 16):
        cnt_v[pl.ds(16 * g, 16)] = zero16

    def chunk(c, carry):
        pltpu.sync_copy(row_hbm.at[pl.ds(ebase + c * HCK, HCK)], ridx_v)

        def gbody(g, carry2):
            rows16 = ridx_v[pl.ds(16 * g, 16)]
            plsc.addupdate_scatter(cnt_v, [rows16], ones16,
                                   mask=jnp.ones((16,), jnp.bool_))
            return carry2

        lax.fori_loop(0, HCK // 16, gbody, 0)
        return carry

    lax.fori_loop(0, EPW // HCK, chunk, 0)
    pltpu.sync_copy(cnt_v.at[pl.ds(0, NHC)],
                    counts_hbm.at[pl.ds(pl.multiple_of(wid * NHC, 8), NHC)])


def _sc_bin_body(row_hbm, col_hbm, bases_hbm, rows_out, cols_out,
                 ridx_v, cidx_v, cur_v, pos_v, sem):
    c_id = lax.axis_index("c")
    s_id = lax.axis_index("s")
    wid = s_id * NC + c_id
    ebase = pl.multiple_of(wid * EPW, EPW)
    iota16 = lax.iota(jnp.int32, 16)
    ohs = [(iota16 == j).astype(jnp.int32) for j in range(16)]

    pltpu.sync_copy(
        bases_hbm.at[pl.ds(pl.multiple_of(wid * NHC, 8), NHC)],
        cur_v.at[pl.ds(0, NHC)])
    lane0 = lax.iota(jnp.int32, 16) == 0
    ones16 = jnp.ones((16,), jnp.int32)

    def chunk(c, carry):
        pltpu.sync_copy(row_hbm.at[pl.ds(ebase + c * HCK, HCK)], ridx_v)
        pltpu.sync_copy(col_hbm.at[pl.ds(ebase + c * HCK, HCK)], cidx_v)

        def gbody(g, carry2):
            kk = lax.div(g, 8)
            m = lax.rem(g, 8)
            rows16 = ridx_v[pl.ds(16 * g, 16)]
            base = plsc.load_gather(cur_v, [rows16])
            rank = jnp.zeros((16,), jnp.int32)
            for s in range(1, 16):
                idx = jnp.where(iota16 >= s, iota16 - s, 0)
                prev = rows16.at[idx].get(mode="promise_in_bounds")
                rank = rank + jnp.where(
                    jnp.logical_and(iota16 >= s, rows16 == prev), 1, 0)
            plsc.addupdate_scatter(cur_v, [rows16], ones16)
            pos_v[kk, pl.ds(16 * m, 16)] = base + rank
            return carry2

        lax.fori_loop(0, HCK // 16, gbody, 0)
        cps = []
        for k in range(HCK // 128):
            cps.append(pltpu.async_copy(cidx_v.at[pl.ds(128 * k, 128)],
                                        cols_out.at[pos_v.at[k]], sem))
            cps.append(pltpu.async_copy(ridx_v.at[pl.ds(128 * k, 128)],
                                        rows_out.at[pos_v.at[k]], sem))
        for cp in cps:
            cp.wait()
        return carry

    lax.fori_loop(0, EPW // HCK, chunk, 0)


def _make_sc_hist():
    mesh = plsc.VectorSubcoreMesh(core_axis_name="c", subcore_axis_name="s",
                                  num_cores=NC, num_subcores=NS)
    return pl.kernel(
        _sc_hist_body,
        out_type=jax.ShapeDtypeStruct((NW * NHC,), jnp.int32),
        mesh=mesh,
        scratch_types=[
            pltpu.VMEM((HCK,), jnp.int32),
            pltpu.VMEM((NH,), jnp.int32),
        ],
        compiler_params=pltpu.CompilerParams(needs_layout_passes=False),
    )


def _make_sc_bin():
    mesh = plsc.VectorSubcoreMesh(core_axis_name="c", subcore_axis_name="s",
                                  num_cores=NC, num_subcores=NS)
    return pl.kernel(
        _sc_bin_body,
        out_type=(jax.ShapeDtypeStruct((ETOT,), jnp.int32),
                  jax.ShapeDtypeStruct((ETOT,), jnp.int32)),
        mesh=mesh,
        scratch_types=[
            pltpu.VMEM((HCK,), jnp.int32),
            pltpu.VMEM((HCK,), jnp.int32),
            pltpu.VMEM((NH,), jnp.int32),
            pltpu.VMEM((HCK // 128, 128), jnp.int32),
            pltpu.SemaphoreType.DMA,
        ],
        compiler_params=pltpu.CompilerParams(needs_layout_passes=False),
    )


def _sc_agg_body(h_hbm, colp_hbm, rowp_hbm, offp_hbm, agg_hbm,
                 hl_v, agg_v, colbuf, cidx, ridx0, ridx1, off_v, sem0, sem1):
    c_id = lax.axis_index("c")
    s_id = lax.axis_index("s")
    wid = s_id * NC + c_id
    nbase = pl.multiple_of(wid * NPW, NPW)
    nrows = jnp.minimum(NPW, N - nbase)

    pltpu.sync_copy(offp_hbm.at[pl.ds(nbase, OFFV)], off_v)
    pltpu.sync_copy(h_hbm.at[pl.ds(nbase, NPW)], hl_v)

    zero16 = jnp.zeros((16,), jnp.float32)

    def zrow(r, carry):
        for k in range(8):
            agg_v[r, pl.ds(16 * k, 16)] = zero16
        return carry

    lax.fori_loop(0, NPW, zrow, 0)

    e0 = _sread(off_v, 0)
    e1 = _sread(off_v, nrows)
    abase = pl.multiple_of(e0 - lax.rem(e0, 8), 8)
    nch = lax.div(e1 - abase + CK - 1, CK)
    sems = (sem0, sem1)
    ridxs = (ridx0, ridx1)

    def issue(c, b):
        cbase = pl.multiple_of(abase + c * CK, 8)
        pltpu.sync_copy(colp_hbm.at[pl.ds(cbase, CK)], cidx.at[b])
        pltpu.sync_copy(rowp_hbm.at[pl.ds(cbase, CK)], ridxs[b].at[pl.ds(0, CK)])
        pltpu.async_copy(h_hbm.at[cidx.at[b]], colbuf.at[b], sems[b])

    def compute(c, b):
        cbase = pl.multiple_of(abase + c * CK, 8)
        cnt = jnp.minimum(CK, e1 - cbase)
        rb = ridxs[b]
        r_lo = jnp.maximum(_sread(rb, 0), nbase)
        r_hi = jnp.minimum(_sread(rb, cnt - 1), nbase + nrows - 1)

        def node_body(r, carry2):
            rl = r - nbase
            lo = jnp.maximum(_sread(off_v, rl), cbase)
            hi = jnp.minimum(_sread(off_v, rl + 1), cbase + cnt)
            hr = tuple(hl_v[rl, pl.ds(16 * k, 16)] for k in range(8))

            def edge_body(e, acc):
                j = e - cbase
                return tuple(
                    acc[k] + jnp.abs(hr[k] - colbuf[b, j, pl.ds(16 * k, 16)])
                    for k in range(8))

            acc0 = tuple(zero16 for _ in range(8))
            acc = lax.fori_loop(lo, hi, edge_body, acc0)
            for k in range(8):
                plsc.addupdate(agg_v.at[rl, pl.ds(16 * k, 16)], acc[k])
            return carry2

        lax.fori_loop(r_lo, r_hi + 1, node_body, 0)

    @pl.when(nch > 0)
    def _():
        issue(0, 0)

    def pair_body(p, carry):
        for b in range(2):
            c = 2 * p + b

            @pl.when(c < nch)
            def _():
                @pl.when(c + 1 < nch)
                def _():
                    issue(c + 1, 1 - b)

                pltpu.make_async_copy(h_hbm.at[cidx.at[b]],
                                      colbuf.at[b], sems[b]).wait()
                compute(c, b)
        return carry

    lax.fori_loop(0, lax.div(nch + 1, 2), pair_body, 0)
    pltpu.sync_copy(agg_v, agg_hbm.at[pl.ds(nbase, NPW)])


def _make_sc_agg():
    mesh = plsc.VectorSubcoreMesh(core_axis_name="c", subcore_axis_name="s",
                                  num_cores=NC, num_subcores=NS)
    return pl.kernel(
        _sc_agg_body,
        out_type=jax.ShapeDtypeStruct((NP, D), jnp.float32),
        mesh=mesh,
        scratch_types=[
            pltpu.VMEM((NPW, D), jnp.float32),
            pltpu.VMEM((NPW, D), jnp.float32),
            pltpu.VMEM((2, CK, D), jnp.float32),
            pltpu.VMEM((2, CK), jnp.int32),
            pltpu.VMEM((CK + 16,), jnp.int32),
            pltpu.VMEM((CK + 16,), jnp.int32),
            pltpu.VMEM((OFFV,), jnp.int32),
            pltpu.SemaphoreType.DMA,
            pltpu.SemaphoreType.DMA,
        ],
    )


BLK = 1280


def _proj_body(x_ref, w_ref, b_ref, o_ref, *, act):
    y = jnp.dot(x_ref[...], w_ref[...], preferred_element_type=jnp.float32)
    y = y + b_ref[...]
    if act == "relu":
        y = jnp.maximum(y, 0.0)
    o_ref[...] = y


def _proj(x, w_t, b, act):
    grid = (NP // BLK,)
    dout = w_t.shape[1]
    return pl.pallas_call(
        functools.partial(_proj_body, act=act),
        grid=grid,
        in_specs=[
            pl.BlockSpec((BLK, D), lambda i: (i, 0)),
            pl.BlockSpec((D, dout), lambda i: (0, 0)),
            pl.BlockSpec((1, dout), lambda i: (0, 0)),
        ],
        out_specs=pl.BlockSpec((BLK, dout), lambda i: (i, 0)),
        out_shape=jax.ShapeDtypeStruct((NP, dout), jnp.float32),
    )(x, w_t, b)


def _gru_body(agg_ref, h_ref, s_ref, wih_ref, whh_ref, bih_ref, bhh_ref,
              o_ref):
    agg = agg_ref[...]
    h = h_ref[...]
    gi = jnp.dot(agg, wih_ref[...], preferred_element_type=jnp.float32)
    gi = gi + bih_ref[...]
    gh = jnp.dot(h, whh_ref[...], preferred_element_type=jnp.float32)
    gh = gh + bhh_ref[...]
    r = jax.nn.sigmoid(gi[:, 0:D] + gh[:, 0:D])
    z = jax.nn.sigmoid(gi[:, D:2 * D] + gh[:, D:2 * D])
    n = jnp.tanh(gi[:, 2 * D:] + r * gh[:, 2 * D:])
    nh = (1.0 - z) * n + z * h
    mask = s_ref[...] > 0
    o_ref[...] = jnp.where(mask, nh, h)


def _gru(agg, h, steps, wih_t, whh_t, bih, bhh):
    grid = (NP // BLK,)
    return pl.pallas_call(
        _gru_body,
        grid=grid,
        in_specs=[
            pl.BlockSpec((BLK, D), lambda i: (i, 0)),
            pl.BlockSpec((BLK, D), lambda i: (i, 0)),
            pl.BlockSpec((BLK, 1), lambda i: (i, 0)),
            pl.BlockSpec((D, 3 * D), lambda i: (0, 0)),
            pl.BlockSpec((D, 3 * D), lambda i: (0, 0)),
            pl.BlockSpec((1, 3 * D), lambda i: (0, 0)),
            pl.BlockSpec((1, 3 * D), lambda i: (0, 0)),
        ],
        out_specs=pl.BlockSpec((BLK, D), lambda i: (i, 0)),
        out_shape=jax.ShapeDtypeStruct((NP, D), jnp.float32),
    )(agg, h, steps, wih_t, whh_t, bih, bhh)


def kernel(x, edge_index, W_in, b_in, Wt1, bt1, Wt2, bt2,
           W_ih, W_hh, b_ih, b_hh, W_out, b_out):
    x = x.astype(jnp.float32)
    row = edge_index[0].astype(jnp.int32)
    col = edge_index[1].astype(jnp.int32)

    rowp32 = jnp.concatenate([row, jnp.full((ETOT - E,), N, jnp.int32)])
    colp32 = jnp.concatenate([col, jnp.zeros((ETOT - E,), jnp.int32)])
    counts = _make_sc_hist()(rowp32).reshape(NW, NHC)
    cnts = counts[:, :N + 1]
    flat = cnts.T.reshape(-1)
    basesF = jnp.cumsum(flat) - flat
    bases2 = basesF.reshape(N + 1, NW).astype(jnp.int32)
    offsets = bases2[:, 0]
    basesT = jnp.concatenate(
        [bases2.T, jnp.zeros((NW, NHC - (N + 1)), jnp.int32)],
        axis=1).reshape(-1)
    rowp, colp = _make_sc_bin()(rowp32, colp32, basesT)
    offp = jnp.concatenate(
        [offsets, jnp.full((OFFPAD - (N + 1),), E, jnp.int32)])

    t = jax.nn.relu(x @ Wt1.T + bt1)
    tau_val = jax.nn.softplus(t @ Wt2.T + bt2).squeeze()
    steps0 = jnp.minimum((1.0 / tau_val).astype(jnp.int32), MAX_REC)
    steps_p = jnp.concatenate(
        [steps0, jnp.zeros((NP - N,), jnp.int32)]).reshape(NP, 1)

    x_p = jnp.concatenate([x, jnp.zeros((NP - N, D), jnp.float32)])

    w_in_t = W_in.T
    wih_t = W_ih.T
    whh_t = W_hh.T
    w_out_t = W_out.T
    b_in2 = b_in.reshape(1, D)
    bih2 = b_ih.reshape(1, 3 * D)
    bhh2 = b_hh.reshape(1, 3 * D)
    b_out2 = b_out.reshape(1, D)

    h0 = _proj(x_p, w_in_t, b_in2, "relu")

    sc_agg = _make_sc_agg()

    def cond(carry):
        _, steps, i = carry
        return jnp.logical_and(i < MAX_REC, jnp.max(steps) > 0)

    def body(carry):
        h, steps, i = carry
        agg = sc_agg(h, colp, rowp, offp)
        h2 = _gru(agg, h, steps, wih_t, whh_t, bih2, bhh2)
        return h2, steps - 1, i + 1

    h, _, _ = lax.while_loop(cond, body, (h0, steps_p, jnp.int32(0)))

    out = _proj(h, w_out_t, b_out2, "none")
    return out[:N]
